# baseline (speedup 1.0000x reference)
import functools

import jax
import jax.numpy as jnp
from jax import lax
from jax.experimental import pallas as pl
from jax.experimental.pallas import tpu as pltpu
from jax.experimental.pallas import tpu_sc as plsc

D_MODEL = 64
SCALE = 8.0
NUM_CORES = 2
NUM_SUBCORES = 16
NUM_WORKERS = NUM_CORES * NUM_SUBCORES
LANES = 16
CHUNK = 512


def _emb_call(n_idx):
    b_per_w = n_idx // NUM_WORKERS
    steps = b_per_w // CHUNK
    mesh = plsc.VectorSubcoreMesh(
        core_axis_name="c", subcore_axis_name="s",
        num_cores=NUM_CORES, num_subcores=NUM_SUBCORES)

    @functools.partial(
        pl.kernel,
        out_type=jax.ShapeDtypeStruct((n_idx, D_MODEL), jnp.float32),
        mesh=mesh,
        compiler_params=pltpu.CompilerParams(use_tc_tiling_on_sc=False),
        scratch_types=[
            pltpu.VMEM((CHUNK,), jnp.int32),
            pltpu.VMEM((CHUNK, D_MODEL), jnp.float32),
            pltpu.SemaphoreType.DMA,
        ],
    )
    def emb(idx_hbm, table_hbm, out_hbm, idx_v, rows_v, sem):
        wid = lax.axis_index("s") * NUM_CORES + lax.axis_index("c")
        base = wid * b_per_w

        def step(g, _):
            off = base + g * CHUNK
            pltpu.sync_copy(idx_hbm.at[pl.ds(off, CHUNK)], idx_v)
            pltpu.async_copy(table_hbm.at[idx_v], rows_v, sem).wait()

            def scale_row(i, _):
                for j in range(D_MODEL // LANES):
                    sl = pl.ds(j * LANES, LANES)
                    rows_v[i, sl] = rows_v[i, sl] * SCALE
                return 0

            lax.fori_loop(0, CHUNK, scale_row, 0)
            pltpu.sync_copy(rows_v, out_hbm.at[pl.ds(off, CHUNK)])
            return 0

        lax.fori_loop(0, steps, step, 0)

    return emb


def kernel(x, table):
    n_idx = x.size
    idx = x.reshape(n_idx).astype(jnp.int32)
    out = _emb_call(n_idx)(idx, table)
    return out.reshape(x.shape + (D_MODEL,))


# speedup vs baseline: 1.1369x; 1.1369x over previous
import functools

import jax
import jax.numpy as jnp
from jax import lax
from jax.experimental import pallas as pl
from jax.experimental.pallas import tpu as pltpu
from jax.experimental.pallas import tpu_sc as plsc

D_MODEL = 64
SCALE = 8.0
NUM_CORES = 2
NUM_SUBCORES = 16
NUM_WORKERS = NUM_CORES * NUM_SUBCORES
LANES = 16
CHUNK = 512


def _emb_call(n_idx):
    b_per_w = n_idx // NUM_WORKERS
    steps = b_per_w // CHUNK
    groups = steps // 2
    mesh = plsc.VectorSubcoreMesh(
        core_axis_name="c", subcore_axis_name="s",
        num_cores=NUM_CORES, num_subcores=NUM_SUBCORES)

    @functools.partial(
        pl.kernel,
        out_type=jax.ShapeDtypeStruct((n_idx, D_MODEL), jnp.float32),
        mesh=mesh,
        compiler_params=pltpu.CompilerParams(use_tc_tiling_on_sc=False),
        scratch_types=[
            pltpu.VMEM((b_per_w,), jnp.int32),
            pltpu.VMEM((CHUNK, D_MODEL), jnp.float32),
            pltpu.VMEM((CHUNK, D_MODEL), jnp.float32),
            pltpu.SemaphoreType.DMA,
            pltpu.SemaphoreType.DMA,
            pltpu.SemaphoreType.DMA,
            pltpu.SemaphoreType.DMA,
        ],
    )
    def emb(idx_hbm, table_hbm, out_hbm, idx_all, rows0, rows1,
            gsem0, gsem1, osem0, osem1):
        wid = lax.axis_index("s") * NUM_CORES + lax.axis_index("c")
        base = wid * b_per_w
        bufs = (rows0, rows1)
        gsems = (gsem0, gsem1)
        osems = (osem0, osem1)

        pltpu.sync_copy(idx_hbm.at[pl.ds(base, b_per_w)], idx_all)

        def gather(g, b):
            return pltpu.make_async_copy(
                table_hbm.at[idx_all.at[pl.ds(g * CHUNK, CHUNK)]],
                bufs[b], gsems[b])

        def writeout(g, b):
            return pltpu.make_async_copy(
                bufs[b], out_hbm.at[pl.ds(base + g * CHUNK, CHUNK)],
                osems[b])

        gather(0, 0).start()

        def group(q, _):
            for b in (0, 1):
                g = q * 2 + b
                buf = bufs[b]
                gather(g, b).wait()

                @pl.when(g >= 1)
                def _():
                    writeout(g - 1, 1 - b).wait()

                @pl.when(g + 1 < steps)
                def _():
                    gather(g + 1, 1 - b).start()

                @plsc.parallel_loop(0, CHUNK, step=1, unroll=4)
                def _(i):
                    for j in range(D_MODEL // LANES):
                        sl = pl.ds(j * LANES, LANES)
                        buf[i, sl] = buf[i, sl] * SCALE

                writeout(g, b).start()
            return 0

        lax.fori_loop(0, groups, group, 0)
        writeout(steps - 1, 1).wait()

    return emb


def kernel(x, table):
    n_idx = x.size
    idx = x.reshape(n_idx).astype(jnp.int32)
    out = _emb_call(n_idx)(idx, table)
    return out.reshape(x.shape + (D_MODEL,))
